# baseline (speedup 1.0000x reference)
import functools

import jax
import jax.numpy as jnp
from jax import lax
from jax.experimental import pallas as pl
from jax.experimental.pallas import tpu as pltpu
from jax.experimental.pallas import tpu_sc as plsc

SEQ = 200
BATCH = 16384
EMBED = 32
IDXW = 128
ROWS_PER_CHUNK = 4
CHUNK = ROWS_PER_CHUNK * IDXW
N_WORKERS = 32
CHUNKS_PER_SEQ = BATCH // CHUNK
N_CHUNKS = SEQ * CHUNKS_PER_SEQ
PER_W = N_CHUNKS // N_WORKERS


def _emb_call():
    mesh = plsc.VectorSubcoreMesh(core_axis_name="c", subcore_axis_name="s")

    @functools.partial(
        pl.kernel,
        mesh=mesh,
        out_type=jax.ShapeDtypeStruct(
            (SEQ, EMBED // 8, BATCH // IDXW, 8, IDXW), jnp.float32
        ),
        scratch_types=[
            [pltpu.VMEM((ROWS_PER_CHUNK, IDXW), jnp.int32) for _ in range(4)],
            [pltpu.VMEM((CHUNK, EMBED), jnp.float32) for _ in range(2)],
            [
                pltpu.VMEM((EMBED // 8, ROWS_PER_CHUNK, 8, IDXW), jnp.float32)
                for _ in range(2)
            ],
            [pltpu.SemaphoreType.DMA for _ in range(4)],
            [pltpu.SemaphoreType.DMA for _ in range(2)],
            [pltpu.SemaphoreType.DMA for _ in range(2)],
        ],
        compiler_params=pltpu.CompilerParams(
            use_tc_tiling_on_sc=False, needs_layout_passes=False
        ),
    )
    def emb(idx_hbm, table_hbm, out_hbm, idx_v, rows_v, tile_v, isem, gsem,
            wsem):
        wid = lax.axis_index("s") * 2 + lax.axis_index("c")
        c_base = wid * PER_W
        iota16 = lax.iota(jnp.int32, 16)

        def start_idx(g, ib):
            c = c_base + jnp.minimum(g, PER_W - 1)
            row = (c // CHUNKS_PER_SEQ) * (BATCH // IDXW) + (
                c % CHUNKS_PER_SEQ
            ) * ROWS_PER_CHUNK
            pltpu.async_copy(
                idx_hbm.at[pl.ds(row, ROWS_PER_CHUNK)], idx_v[ib], isem[ib]
            )

        def wait_idx(ib):
            pltpu.make_async_copy(
                idx_hbm.at[pl.ds(0, ROWS_PER_CHUNK)], idx_v[ib], isem[ib]
            ).wait()

        def start_gathers(ib, b):
            for rr in range(ROWS_PER_CHUNK):
                pltpu.async_copy(
                    table_hbm.at[idx_v[ib].at[rr]],
                    rows_v[b].at[pl.ds(rr * IDXW, IDXW)],
                    gsem[b],
                )

        def wait_gathers(b):
            pltpu.make_async_copy(
                table_hbm.at[pl.ds(0, CHUNK)], rows_v[b], gsem[b]
            ).wait()

        def transpose(b):
            rv, tv = rows_v[b], tile_v[b]

            @plsc.parallel_loop(0, (CHUNK // 16) * EMBED, unroll=8)
            def _tr(p):
                g = p // EMBED
                e = p % EMBED
                row_idx = iota16 + g * 16
                col_idx = jnp.broadcast_to(e, (16,))
                vals = plsc.load_gather(rv, [row_idx, col_idx])
                tv[e // 8, g // 8, e % 8, pl.ds((g % 8) * 16, 16)] = vals

        def start_wb(g, b):
            c = c_base + g
            j = c // CHUNKS_PER_SEQ
            ti0 = (c % CHUNKS_PER_SEQ) * ROWS_PER_CHUNK
            pltpu.async_copy(
                tile_v[b],
                out_hbm.at[j, :, pl.ds(ti0, ROWS_PER_CHUNK)],
                wsem[b],
            )

        def wait_wb(b):
            pltpu.make_async_copy(
                tile_v[b],
                out_hbm.at[0, :, pl.ds(0, ROWS_PER_CHUNK)],
                wsem[b],
            ).wait()

        def body(g, u4, u2, first=False, second=False):
            start_idx(g + 2, (u4 + 2) % 4)
            wait_idx(u4)
            start_gathers(u4, u2)
            wait_gathers(1 - u2)
            if not (first or second):
                wait_wb(1 - u2)
            transpose(1 - u2)
            start_wb(g - 1, 1 - u2)

        for t in range(3):
            start_idx(jnp.int32(t), t)
        wait_idx(0)
        start_gathers(0, 0)

        body(jnp.int32(1), 1, 1, first=True)
        body(jnp.int32(2), 2, 0, second=True)

        def loop_body(kk, carry):
            for u in range(4):
                g = kk * 4 + 3 + u
                body(g, (3 + u) % 4, (3 + u) % 2)
            return carry

        lax.fori_loop(0, (PER_W - 4) // 4, loop_body, jnp.int32(0))

        body(jnp.int32(PER_W - 1), (PER_W - 1) % 4, (PER_W - 1) % 2)

        wait_gathers((PER_W - 1) % 2)
        wait_wb((PER_W - 1) % 2)
        transpose((PER_W - 1) % 2)
        start_wb(jnp.int32(PER_W - 1), (PER_W - 1) % 2)
        wait_wb((PER_W - 2) % 2)
        wait_wb((PER_W - 1) % 2)
        wait_idx(PER_W % 4)
        wait_idx((PER_W + 1) % 4)

    return emb


def kernel(x, table):
    idx2d = x.T.reshape((BATCH * SEQ) // IDXW, IDXW)
    L = _emb_call()(idx2d, table)
    return L.transpose(2, 4, 0, 1, 3).reshape(BATCH, SEQ, EMBED)


# speedup vs baseline: 2.1927x; 2.1927x over previous
import functools

import jax
import jax.numpy as jnp
from jax import lax
from jax.experimental import pallas as pl
from jax.experimental.pallas import tpu as pltpu
from jax.experimental.pallas import tpu_sc as plsc

SEQ = 200
BATCH = 16384
EMBED = 32
IDXW = 128
ROWS_PER_CHUNK = 4
CHUNK = ROWS_PER_CHUNK * IDXW
N_WORKERS = 32
CHUNKS_PER_SEQ = BATCH // CHUNK
N_CHUNKS = SEQ * CHUNKS_PER_SEQ
PER_W = N_CHUNKS // N_WORKERS


def _emb_call():
    mesh = plsc.VectorSubcoreMesh(core_axis_name="c", subcore_axis_name="s")

    @functools.partial(
        pl.kernel,
        mesh=mesh,
        out_type=jax.ShapeDtypeStruct(
            (SEQ, EMBED // 8, BATCH // IDXW, 8, IDXW), jnp.float32
        ),
        scratch_types=[
            [pltpu.VMEM((ROWS_PER_CHUNK, IDXW), jnp.int32) for _ in range(4)],
            [pltpu.VMEM((CHUNK, EMBED), jnp.float32) for _ in range(2)],
            [pltpu.VMEM((160, 133), jnp.float32) for _ in range(2)],
            [pltpu.SemaphoreType.DMA for _ in range(4)],
            [pltpu.SemaphoreType.DMA for _ in range(2)],
            [pltpu.SemaphoreType.DMA for _ in range(2)],
        ],
        compiler_params=pltpu.CompilerParams(
            use_tc_tiling_on_sc=False, needs_layout_passes=False
        ),
    )
    def emb(idx_hbm, table_hbm, out_hbm, idx_v, rows_v, tile_v, isem, gsem,
            wsem):
        wid = lax.axis_index("s") * 2 + lax.axis_index("c")
        c_base = wid * PER_W
        iota16 = lax.iota(jnp.int32, 16)

        def start_idx(g, ib):
            c = c_base + jnp.minimum(g, PER_W - 1)
            row = (c // CHUNKS_PER_SEQ) * (BATCH // IDXW) + (
                c % CHUNKS_PER_SEQ
            ) * ROWS_PER_CHUNK
            pltpu.async_copy(
                idx_hbm.at[pl.ds(row, ROWS_PER_CHUNK)], idx_v[ib], isem[ib]
            )

        def wait_idx(ib):
            pltpu.make_async_copy(
                idx_hbm.at[pl.ds(0, ROWS_PER_CHUNK)], idx_v[ib], isem[ib]
            ).wait()

        def start_gathers(ib, b):
            for rr in range(ROWS_PER_CHUNK):
                pltpu.async_copy(
                    table_hbm.at[idx_v[ib].at[rr]],
                    rows_v[b].at[pl.ds(rr * IDXW, IDXW)],
                    gsem[b],
                )

        def wait_gathers(b):
            pltpu.make_async_copy(
                table_hbm.at[pl.ds(0, CHUNK)], rows_v[b], gsem[b]
            ).wait()

        rowc_lo = (iota16 // 8) * 40 + iota16 % 8
        rowc_hi = rowc_lo + 80

        def transpose(b):
            rv, tv = rows_v[b], tile_v[b]

            @plsc.parallel_loop(0, CHUNK, unroll=8)
            def _tr(t):
                tjj = t // IDXW
                cc = t % IDXW
                row1 = rowc_lo + tjj * 10
                row2 = rowc_hi + tjj * 10
                colv = jnp.broadcast_to(cc, (16,))
                v1 = rv[t, pl.ds(0, 16)]
                v2 = rv[t, pl.ds(16, 16)]
                plsc.store_scatter(tv, [row1, colv], v1)
                plsc.store_scatter(tv, [row2, colv], v2)

        def start_wb(g, b):
            c = c_base + g
            j = c // CHUNKS_PER_SEQ
            ti0 = (c % CHUNKS_PER_SEQ) * ROWS_PER_CHUNK
            for te in range(EMBED // 8):
                for tj in range(ROWS_PER_CHUNK):
                    pltpu.async_copy(
                        tile_v[b].at[
                            pl.ds(te * 40 + tj * 10, 8), pl.ds(0, IDXW)
                        ],
                        out_hbm.at[j, te, ti0 + tj],
                        wsem[b],
                    )

        def wait_wb(b):
            for _ in range(EMBED // 8 * ROWS_PER_CHUNK):
                pltpu.make_async_copy(
                    tile_v[b].at[pl.ds(0, 8), pl.ds(0, IDXW)],
                    out_hbm.at[0, 0, 0],
                    wsem[b],
                ).wait()

        def body(g, u4, u2, first=False, second=False):
            start_idx(g + 2, (u4 + 2) % 4)
            wait_idx(u4)
            start_gathers(u4, u2)
            wait_gathers(1 - u2)
            if not (first or second):
                wait_wb(1 - u2)
            transpose(1 - u2)
            start_wb(g - 1, 1 - u2)

        for t in range(3):
            start_idx(jnp.int32(t), t)
        wait_idx(0)
        start_gathers(0, 0)

        body(jnp.int32(1), 1, 1, first=True)
        body(jnp.int32(2), 2, 0, second=True)

        def loop_body(kk, carry):
            for u in range(4):
                g = kk * 4 + 3 + u
                body(g, (3 + u) % 4, (3 + u) % 2)
            return carry

        lax.fori_loop(0, (PER_W - 4) // 4, loop_body, jnp.int32(0))

        body(jnp.int32(PER_W - 1), (PER_W - 1) % 4, (PER_W - 1) % 2)

        wait_gathers((PER_W - 1) % 2)
        wait_wb((PER_W - 1) % 2)
        transpose((PER_W - 1) % 2)
        start_wb(jnp.int32(PER_W - 1), (PER_W - 1) % 2)
        wait_wb((PER_W - 2) % 2)
        wait_wb((PER_W - 1) % 2)
        wait_idx(PER_W % 4)
        wait_idx((PER_W + 1) % 4)

    return emb


def kernel(x, table):
    idx2d = x.T.reshape((BATCH * SEQ) // IDXW, IDXW)
    L = _emb_call()(idx2d, table)
    return L.transpose(2, 4, 0, 1, 3).reshape(BATCH, SEQ, EMBED)
